# Initial kernel scaffold; baseline (speedup 1.0000x reference)
import jax
import jax.numpy as jnp
from jax.experimental import pallas as pl


def kernel(x, edge_index, W_rel, b_rel, W_root):
    raise NotImplementedError("write your pallas kernel here")


# speedup vs baseline: 34.3907x; 34.3907x over previous
import functools

import jax
import jax.numpy as jnp
from jax import lax
from jax.experimental import pallas as pl
from jax.experimental.pallas import tpu as pltpu
from jax.experimental.pallas import tpu_sc as plsc

N_NODES = 10000
D = 128
N_EDGES = 320000

_NC = 2
_NS = 16
_NW = _NC * _NS
_L = 16
_E_PER = N_EDGES // _NW
_ROW_BLK = 2000


def _matvec_body(x_ref, wrel_ref, wroot_ref, b_ref, srel_ref, base_ref):
    xb = x_ref[...]
    srel_ref[...] = jnp.dot(xb, wrel_ref[...], preferred_element_type=jnp.float32)
    base_ref[...] = (
        jnp.dot(xb, wroot_ref[...], preferred_element_type=jnp.float32) + b_ref[0, 0]
    )


def _edge_body(srel_hbm, edge_hbm, out_hbm, srel_v, src_v, dst_v, acc_v):
    wid = lax.axis_index("s") * _NC + lax.axis_index("c")
    e0 = pl.multiple_of(wid * _E_PER, 8)
    pltpu.sync_copy(srel_hbm, srel_v)
    pltpu.sync_copy(edge_hbm.at[pl.ds(e0, _E_PER)], src_v)
    pltpu.sync_copy(edge_hbm.at[pl.ds(N_EDGES + e0, _E_PER)], dst_v)

    zeros = jnp.zeros((_L,), jnp.float32)

    def zero_step(i, carry):
        acc_v[pl.ds(i * _L, _L)] = zeros
        return carry

    lax.fori_loop(0, N_NODES // _L, zero_step, 0)

    def edge_step(i, carry):
        sl = pl.ds(i * _L, _L)
        si = src_v[sl]
        di = dst_v[sl]
        vals = plsc.load_gather(srel_v, [si])
        plsc.addupdate_scatter(acc_v, [di], vals)
        return carry

    lax.fori_loop(0, _E_PER // _L, edge_step, 0)

    pltpu.sync_copy(acc_v, out_hbm.at[wid])


_edge_kernel = functools.partial(
    pl.kernel,
    mesh=plsc.VectorSubcoreMesh(core_axis_name="c", subcore_axis_name="s"),
    compiler_params=pltpu.CompilerParams(needs_layout_passes=False),
    out_type=jax.ShapeDtypeStruct((_NW, N_NODES), jnp.float32),
    scratch_types=[
        pltpu.VMEM((N_NODES,), jnp.float32),
        pltpu.VMEM((_E_PER,), jnp.int32),
        pltpu.VMEM((_E_PER,), jnp.int32),
        pltpu.VMEM((N_NODES,), jnp.float32),
    ],
)(_edge_body)


def _combine_body(p_ref, base_ref, out_ref):
    out_ref[...] = jnp.sum(p_ref[...], axis=0, keepdims=True) + base_ref[...]


def kernel(x, edge_index, W_rel, b_rel, W_root):
    edges = edge_index.astype(jnp.int32).reshape(2 * N_EDGES)
    srel, base = pl.pallas_call(
        _matvec_body,
        grid=(N_NODES // _ROW_BLK,),
        in_specs=[
            pl.BlockSpec((_ROW_BLK, D), lambda i: (i, 0)),
            pl.BlockSpec((D, 1), lambda i: (0, 0)),
            pl.BlockSpec((D, 1), lambda i: (0, 0)),
            pl.BlockSpec((1, 1), lambda i: (0, 0)),
        ],
        out_specs=[
            pl.BlockSpec((_ROW_BLK, 1), lambda i: (i, 0)),
            pl.BlockSpec((_ROW_BLK, 1), lambda i: (i, 0)),
        ],
        out_shape=[
            jax.ShapeDtypeStruct((N_NODES, 1), jnp.float32),
            jax.ShapeDtypeStruct((N_NODES, 1), jnp.float32),
        ],
    )(x, W_rel, W_root, b_rel.reshape(1, 1))

    partials = _edge_kernel(srel.reshape(N_NODES), edges)

    out_row = pl.pallas_call(
        _combine_body,
        out_shape=jax.ShapeDtypeStruct((1, N_NODES), jnp.float32),
    )(partials, base.reshape(1, N_NODES))
    return out_row.reshape(N_NODES, 1)


# speedup vs baseline: 36.3104x; 1.0558x over previous
import functools

import jax
import jax.numpy as jnp
from jax import lax
from jax.experimental import pallas as pl
from jax.experimental.pallas import tpu as pltpu
from jax.experimental.pallas import tpu_sc as plsc

N_NODES = 10000
D = 128
N_EDGES = 320000

_NC = 2
_NS = 16
_NW = _NC * _NS
_L = 16
_E_PER = N_EDGES // _NW
_ROW_BLK = 2000


def _matvec_body(x_ref, wrel_ref, wroot_ref, b_ref, srel_ref, base_ref):
    xb = x_ref[...]
    srel_ref[...] = jnp.dot(
        xb, wrel_ref[...], preferred_element_type=jnp.float32,
        precision=jax.lax.Precision.HIGHEST,
    )
    base_ref[...] = (
        jnp.dot(
            xb, wroot_ref[...], preferred_element_type=jnp.float32,
            precision=jax.lax.Precision.HIGHEST,
        )
        + b_ref[0, 0]
    )


def _edge_body(zeros_hbm, srel_hbm, edge_hbm, out_hbm, srel_v, src_v, dst_v, acc_v, sem):
    wid = lax.axis_index("s") * _NC + lax.axis_index("c")
    e0 = pl.multiple_of(wid * _E_PER, 8)
    cps = [
        pltpu.async_copy(zeros_hbm, acc_v, sem),
        pltpu.async_copy(srel_hbm, srel_v, sem),
        pltpu.async_copy(edge_hbm.at[pl.ds(e0, _E_PER)], src_v, sem),
        pltpu.async_copy(edge_hbm.at[pl.ds(N_EDGES + e0, _E_PER)], dst_v, sem),
    ]
    for cp in cps:
        cp.wait()

    @plsc.parallel_loop(0, _E_PER // _L, unroll=25)
    def edge_step(i):
        sl = pl.ds(i * _L, _L)
        vals = plsc.load_gather(srel_v, [src_v[sl]])
        plsc.addupdate_scatter(acc_v, [dst_v[sl]], vals)

    pltpu.sync_copy(acc_v, out_hbm.at[wid])


_edge_kernel = functools.partial(
    pl.kernel,
    mesh=plsc.VectorSubcoreMesh(core_axis_name="c", subcore_axis_name="s"),
    compiler_params=pltpu.CompilerParams(needs_layout_passes=False),
    out_type=jax.ShapeDtypeStruct((_NW, N_NODES), jnp.float32),
    scratch_types=[
        pltpu.VMEM((N_NODES,), jnp.float32),
        pltpu.VMEM((_E_PER,), jnp.int32),
        pltpu.VMEM((_E_PER,), jnp.int32),
        pltpu.VMEM((N_NODES,), jnp.float32),
        pltpu.SemaphoreType.DMA,
    ],
)(_edge_body)


def _combine_body(p_ref, base_ref, out_ref):
    out_ref[...] = jnp.sum(p_ref[...], axis=0, keepdims=True) + base_ref[...]


def kernel(x, edge_index, W_rel, b_rel, W_root):
    edges = edge_index.astype(jnp.int32).reshape(2 * N_EDGES)
    srel, base = pl.pallas_call(
        _matvec_body,
        grid=(N_NODES // _ROW_BLK,),
        in_specs=[
            pl.BlockSpec((_ROW_BLK, D), lambda i: (i, 0)),
            pl.BlockSpec((D, 1), lambda i: (0, 0)),
            pl.BlockSpec((D, 1), lambda i: (0, 0)),
            pl.BlockSpec((1, 1), lambda i: (0, 0)),
        ],
        out_specs=[
            pl.BlockSpec((_ROW_BLK, 1), lambda i: (i, 0)),
            pl.BlockSpec((_ROW_BLK, 1), lambda i: (i, 0)),
        ],
        out_shape=[
            jax.ShapeDtypeStruct((N_NODES, 1), jnp.float32),
            jax.ShapeDtypeStruct((N_NODES, 1), jnp.float32),
        ],
    )(x, W_rel, W_root, b_rel.reshape(1, 1))

    partials = _edge_kernel(
        jnp.zeros((N_NODES,), jnp.float32), srel.reshape(N_NODES), edges
    )

    out_row = pl.pallas_call(
        _combine_body,
        out_shape=jax.ShapeDtypeStruct((1, N_NODES), jnp.float32),
    )(partials, base.reshape(1, N_NODES))
    return out_row.reshape(N_NODES, 1)


# speedup vs baseline: 52.0513x; 1.4335x over previous
import functools

import jax
import jax.numpy as jnp
from jax import lax
from jax.experimental import pallas as pl
from jax.experimental.pallas import tpu as pltpu
from jax.experimental.pallas import tpu_sc as plsc

N_NODES = 10000
D = 128
N_EDGES = 320000

_NC = 2
_NS = 16
_NW = _NC * _NS
_L = 16
_ROW_BLK = 2000

_CHUNK = 128
_BASE_CHUNKS = 78
_MAX_EDGES = (_BASE_CHUNKS + 1) * _CHUNK


def _matvec_body(x_ref, wrel_ref, wroot_ref, b_ref, srel_ref, base_ref):
    xb = x_ref[...]
    srel_ref[...] = jnp.sum(xb * wrel_ref[...], axis=1)
    base_ref[...] = jnp.sum(xb * wroot_ref[...], axis=1) + b_ref[0, 0]


def _edge_body(srel_hbm, edge_hbm, out_hbm, srel_v, edges_v, acc_v, sem):
    wid = lax.axis_index("s") * _NC + lax.axis_index("c")
    extra = jnp.maximum(wid - 28, 0)
    c0 = pl.multiple_of((wid * _BASE_CHUNKS + extra) * _CHUNK, _CHUNK)
    nvec = (_BASE_CHUNKS * _CHUNK) // _L + jnp.where(wid >= 28, 8, 0)

    cps = [
        pltpu.async_copy(srel_hbm, srel_v, sem),
        pltpu.async_copy(edge_hbm.at[:, pl.ds(c0, _MAX_EDGES)], edges_v, sem),
    ]

    zero16 = jnp.zeros((_L,), jnp.float32)

    @plsc.parallel_loop(0, N_NODES // _L, unroll=8)
    def zero_step(i):
        acc_v[pl.ds(i * _L, _L)] = zero16

    for cp in cps:
        cp.wait()

    @plsc.parallel_loop(0, nvec, unroll=8)
    def edge_step(k):
        sl = pl.ds(k * _L, _L)
        vals = plsc.load_gather(srel_v, [edges_v[0, sl]])
        plsc.addupdate_scatter(acc_v, [edges_v[1, sl]], vals)

    pltpu.sync_copy(acc_v, out_hbm.at[wid])


_edge_kernel = functools.partial(
    pl.kernel,
    mesh=plsc.VectorSubcoreMesh(core_axis_name="c", subcore_axis_name="s"),
    compiler_params=pltpu.CompilerParams(needs_layout_passes=False),
    out_type=jax.ShapeDtypeStruct((_NW, N_NODES), jnp.float32),
    scratch_types=[
        pltpu.VMEM((N_NODES,), jnp.float32),
        pltpu.VMEM((2, _MAX_EDGES), jnp.int32),
        pltpu.VMEM((N_NODES,), jnp.float32),
        pltpu.SemaphoreType.DMA,
    ],
)(_edge_body)


def _combine_body(p_ref, base_ref, out_ref):
    out_ref[...] = jnp.sum(p_ref[...], axis=0, keepdims=True) + base_ref[...][None, :]


def kernel(x, edge_index, W_rel, b_rel, W_root):
    edges = edge_index.astype(jnp.int32)
    srel, base = pl.pallas_call(
        _matvec_body,
        out_shape=[
            jax.ShapeDtypeStruct((N_NODES,), jnp.float32),
            jax.ShapeDtypeStruct((N_NODES,), jnp.float32),
        ],
    )(x, W_rel.reshape(1, D), W_root.reshape(1, D), b_rel.reshape(1, 1))

    partials = _edge_kernel(srel, edges)

    out_row = pl.pallas_call(
        _combine_body,
        out_shape=jax.ShapeDtypeStruct((1, N_NODES), jnp.float32),
    )(partials, base)
    return out_row.reshape(N_NODES, 1)


# speedup vs baseline: 60.9345x; 1.1707x over previous
import functools

import jax
import jax.numpy as jnp
from jax import lax
from jax.experimental import pallas as pl
from jax.experimental.pallas import tpu as pltpu
from jax.experimental.pallas import tpu_sc as plsc

N_NODES = 10000
D = 128
N_EDGES = 320000

_NC = 2
_NS = 16
_NW = _NC * _NS
_L = 16
_ROW_BLK = 2000

_CHUNK = 128
_BASE_CHUNKS = 78
_MAX_EDGES = (_BASE_CHUNKS + 1) * _CHUNK


def _matvec_body(x_ref, wrel_ref, wroot_ref, b_ref, srel_ref, base_ref):
    xb = x_ref[...]
    dn = (((1,), (1,)), ((), ()))
    srel_ref[...] = jax.lax.dot_general(
        wrel_ref[...], xb, dn, preferred_element_type=jnp.float32
    )
    base_ref[...] = (
        jax.lax.dot_general(wroot_ref[...], xb, dn, preferred_element_type=jnp.float32)
        + b_ref[0, 0]
    )


def _edge_body(srel_hbm, edge_hbm, out_hbm, srel_v, edges_v, acc_v, sem):
    wid = lax.axis_index("s") * _NC + lax.axis_index("c")
    extra = jnp.maximum(wid - 28, 0)
    c0 = pl.multiple_of((wid * _BASE_CHUNKS + extra) * _CHUNK, _CHUNK)
    nvec = (_BASE_CHUNKS * _CHUNK) // _L + jnp.where(wid >= 28, 8, 0)

    cps = [
        pltpu.async_copy(srel_hbm, srel_v, sem),
        pltpu.async_copy(edge_hbm.at[:, pl.ds(c0, _MAX_EDGES)], edges_v, sem),
    ]

    zero16 = jnp.zeros((_L,), jnp.float32)

    @plsc.parallel_loop(0, N_NODES // _L, unroll=8)
    def zero_step(i):
        acc_v[pl.ds(i * _L, _L)] = zero16

    for cp in cps:
        cp.wait()

    @plsc.parallel_loop(0, nvec, unroll=8)
    def edge_step(k):
        sl = pl.ds(k * _L, _L)
        vals = plsc.load_gather(srel_v, [edges_v[0, sl]])
        plsc.addupdate_scatter(acc_v, [edges_v[1, sl]], vals)

    pltpu.sync_copy(acc_v, out_hbm.at[wid])


_edge_kernel = functools.partial(
    pl.kernel,
    mesh=plsc.VectorSubcoreMesh(core_axis_name="c", subcore_axis_name="s"),
    compiler_params=pltpu.CompilerParams(needs_layout_passes=False),
    out_type=jax.ShapeDtypeStruct((_NW, N_NODES), jnp.float32),
    scratch_types=[
        pltpu.VMEM((N_NODES,), jnp.float32),
        pltpu.VMEM((2, _MAX_EDGES), jnp.int32),
        pltpu.VMEM((N_NODES,), jnp.float32),
        pltpu.SemaphoreType.DMA,
    ],
)(_edge_body)


def _combine_body(p_ref, base_ref, out_ref):
    out_ref[...] = jnp.sum(p_ref[...], axis=0, keepdims=True) + base_ref[...]


def kernel(x, edge_index, W_rel, b_rel, W_root):
    edges = edge_index.astype(jnp.int32)
    srel, base = pl.pallas_call(
        _matvec_body,
        out_shape=[
            jax.ShapeDtypeStruct((1, N_NODES), jnp.float32),
            jax.ShapeDtypeStruct((1, N_NODES), jnp.float32),
        ],
    )(x, W_rel.reshape(1, D), W_root.reshape(1, D), b_rel.reshape(1, 1))

    partials = _edge_kernel(srel.reshape(N_NODES), edges)

    out_row = pl.pallas_call(
        _combine_body,
        out_shape=jax.ShapeDtypeStruct((1, N_NODES), jnp.float32),
    )(partials, base)
    return out_row.reshape(N_NODES, 1)
